# Initial kernel scaffold; baseline (speedup 1.0000x reference)
import jax
import jax.numpy as jnp
from jax.experimental import pallas as pl


def kernel(x, keep_coeff, mix_coeff):
    raise NotImplementedError("write your pallas kernel here")


# speedup vs baseline: 6.2573x; 6.2573x over previous
import numpy as np
import jax
import jax.numpy as jnp
from jax.experimental import pallas as pl
from math import factorial, sqrt

_METADATA_IN = [128, 64, 32]
_LMAX_IN = 2
_LMAX_OUT = 4
_IN_OFF = [0, 128, 320]


def _cg_complex(j1, m1, j2, m2, j3, m3):
    if m1 + m2 != m3:
        return 0.0
    if not (abs(j1 - j2) <= j3 <= j1 + j2):
        return 0.0
    f = factorial
    pre = ((2 * j3 + 1) * f(j1 + j2 - j3) * f(j1 - j2 + j3) * f(-j1 + j2 + j3) / f(j1 + j2 + j3 + 1)) ** 0.5
    pre *= (f(j3 + m3) * f(j3 - m3) * f(j1 + m1) * f(j1 - m1) * f(j2 + m2) * f(j2 - m2)) ** 0.5
    kmin = max(0, j2 - j3 - m1, j1 - j3 + m2)
    kmax = min(j1 + j2 - j3, j1 - m1, j2 + m2)
    s = 0.0
    for k in range(kmin, kmax + 1):
        s += (-1) ** k / (f(k) * f(j1 + j2 - j3 - k) * f(j1 - m1 - k) * f(j2 + m2 - k) * f(j3 - j2 + m1 + k) * f(j3 - j1 - m2 + k))
    return pre * s


def _u_matrix(l):
    U = np.zeros((2 * l + 1, 2 * l + 1), dtype=np.complex128)
    U[l, l] = 1.0
    for m in range(1, l + 1):
        U[l + m, l + m] = (-1) ** m / sqrt(2.0)
        U[l + m, l - m] = 1.0 / sqrt(2.0)
        U[l - m, l - m] = 1j / sqrt(2.0)
        U[l - m, l + m] = -1j * (-1) ** m / sqrt(2.0)
    return U


def _real_cg(l1, l2, l3):
    Cc = np.zeros((2 * l3 + 1, 2 * l1 + 1, 2 * l2 + 1), dtype=np.complex128)
    for m3 in range(-l3, l3 + 1):
        for m1 in range(-l1, l1 + 1):
            m2 = m3 - m1
            if abs(m2) <= l2:
                Cc[m3 + l3, m1 + l1, m2 + l2] = _cg_complex(l1, m1, l2, m2, l3, m3)
    U1, U2, U3 = _u_matrix(l1), _u_matrix(l2), _u_matrix(l3)
    Cr = np.einsum('Km,kij,Ii,Jj->KIJ', U3, Cc, U1.conj(), U2.conj(), optimize=True)
    if np.abs(Cr.imag).max() > np.abs(Cr.real).max():
        return np.ascontiguousarray(Cr.imag)
    return np.ascontiguousarray(Cr.real)


def _build_terms():
    couplings = []
    for lout in range(_LMAX_OUT + 1):
        for l1 in range(_LMAX_IN + 1):
            for l2 in range(_LMAX_IN + 1):
                if abs(l1 - l2) <= lout <= l1 + l2:
                    deg = min(_METADATA_IN[l1], _METADATA_IN[l2])
                    if deg > 0:
                        couplings.append((lout, l1, l2, deg))
    metadata_cg = [0] * (_LMAX_OUT + 1)
    metadata_out = [0] * (_LMAX_OUT + 1)
    for lo, _, _, d in couplings:
        metadata_cg[lo] += d
        metadata_out[lo] = max(metadata_out[lo], d)
    base = np.concatenate([[0], np.cumsum(metadata_cg)[:-1]]).astype(int)
    within = [0] * (_LMAX_OUT + 1)
    terms = []
    for lo, l1, l2, deg in couplings:
        C = _real_cg(l1, l2, lo)
        nz = []
        for k in range(C.shape[0]):
            for i in range(C.shape[1]):
                for j in range(C.shape[2]):
                    v = float(C[k, i, j])
                    if abs(v) > 1e-14:
                        nz.append((k, i, j, v))
        mc_off = int(base[lo]) + within[lo]
        terms.append((lo, l1, l2, deg, mc_off, nz))
        within[lo] += deg
    return terms, metadata_out


_TERMS, _META_OUT = _build_terms()
_DIM_IN = sum((2 * l + 1) * n for l, n in enumerate(_METADATA_IN))
_DIM_OUT = sum((2 * lo + 1) * _META_OUT[lo] for lo in range(_LMAX_OUT + 1))


def _body(xt_ref, kc_ref, mc_ref, ot_ref):
    xt = xt_ref[...]
    kc = kc_ref[...]
    mc = mc_ref[...]
    nb = xt.shape[1]

    def xseg(l, m, w):
        base = _IN_OFF[l] + m * _METADATA_IN[l]
        return xt[base:base + w, :]

    prods = {}

    def prod(l1, l2, i, j, w):
        key = (l1, l2, i, j) if (l1, l2, i, j) <= (l2, l1, j, i) else (l2, l1, j, i)
        if key not in prods:
            prods[key] = xseg(key[0], key[2], w) * xseg(key[1], key[3], w)
        return prods[key]

    acc = {}

    def add(lo, k, w, arr):
        d = acc.setdefault((lo, k), {})
        d[w] = d[w] + arr if w in d else arr

    ch = 0
    for l, nc in enumerate(_METADATA_IN):
        cp = min(nc, _META_OUT[l])
        kcv = kc[ch:ch + cp, :]
        for m in range(2 * l + 1):
            add(l, m, cp, xseg(l, m, cp) * kcv)
        ch += nc

    for (lo, l1, l2, deg, mc_off, nzs) in _TERMS:
        for (k, i, j, v) in nzs:
            tv = mc[mc_off:mc_off + deg, :] * (0.5 * v)
            add(lo, k, deg, prod(l1, l2, i, j, deg) * tv)

    blocks = []
    for lo in range(_LMAX_OUT + 1):
        W = _META_OUT[lo]
        for k in range(2 * lo + 1):
            d = acc.get((lo, k), {})
            widths = sorted(d, reverse=True)
            if widths and widths[0] == W:
                cur = d[W]
                widths = widths[1:]
            else:
                cur = jnp.zeros((W, nb), xt.dtype)
            for w in widths:
                cur = jnp.concatenate([cur[:w, :] + d[w], cur[w:, :]], axis=0)
            blocks.append(cur)
    ot_ref[...] = jnp.concatenate(blocks, axis=0)


def kernel(x, keep_coeff, mix_coeff):
    n = x.shape[0]
    xt = x.T
    kc = keep_coeff.reshape(-1, 1)
    mc = mix_coeff.reshape(-1, 1)
    NB = 1024
    grid = (n // NB,)
    ot = pl.pallas_call(
        _body,
        grid=grid,
        in_specs=[
            pl.BlockSpec((_DIM_IN, NB), lambda i: (0, i)),
            pl.BlockSpec((224, 1), lambda i: (0, 0)),
            pl.BlockSpec((864, 1), lambda i: (0, 0)),
        ],
        out_specs=pl.BlockSpec((_DIM_OUT, NB), lambda i: (0, i)),
        out_shape=jax.ShapeDtypeStruct((_DIM_OUT, n), x.dtype),
    )(xt, kc, mc)
    return ot.T


# speedup vs baseline: 8.4962x; 1.3578x over previous
import numpy as np
import jax
import jax.numpy as jnp
from jax.experimental import pallas as pl
from math import factorial, sqrt

_METADATA_IN = [128, 64, 32]
_LMAX_IN = 2
_LMAX_OUT = 4
_IN_OFF = [0, 128, 320]


def _cg_complex(j1, m1, j2, m2, j3, m3):
    if m1 + m2 != m3:
        return 0.0
    if not (abs(j1 - j2) <= j3 <= j1 + j2):
        return 0.0
    f = factorial
    pre = ((2 * j3 + 1) * f(j1 + j2 - j3) * f(j1 - j2 + j3) * f(-j1 + j2 + j3) / f(j1 + j2 + j3 + 1)) ** 0.5
    pre *= (f(j3 + m3) * f(j3 - m3) * f(j1 + m1) * f(j1 - m1) * f(j2 + m2) * f(j2 - m2)) ** 0.5
    kmin = max(0, j2 - j3 - m1, j1 - j3 + m2)
    kmax = min(j1 + j2 - j3, j1 - m1, j2 + m2)
    s = 0.0
    for k in range(kmin, kmax + 1):
        s += (-1) ** k / (f(k) * f(j1 + j2 - j3 - k) * f(j1 - m1 - k) * f(j2 + m2 - k) * f(j3 - j2 + m1 + k) * f(j3 - j1 - m2 + k))
    return pre * s


def _u_matrix(l):
    U = np.zeros((2 * l + 1, 2 * l + 1), dtype=np.complex128)
    U[l, l] = 1.0
    for m in range(1, l + 1):
        U[l + m, l + m] = (-1) ** m / sqrt(2.0)
        U[l + m, l - m] = 1.0 / sqrt(2.0)
        U[l - m, l - m] = 1j / sqrt(2.0)
        U[l - m, l + m] = -1j * (-1) ** m / sqrt(2.0)
    return U


def _real_cg(l1, l2, l3):
    Cc = np.zeros((2 * l3 + 1, 2 * l1 + 1, 2 * l2 + 1), dtype=np.complex128)
    for m3 in range(-l3, l3 + 1):
        for m1 in range(-l1, l1 + 1):
            m2 = m3 - m1
            if abs(m2) <= l2:
                Cc[m3 + l3, m1 + l1, m2 + l2] = _cg_complex(l1, m1, l2, m2, l3, m3)
    U1, U2, U3 = _u_matrix(l1), _u_matrix(l2), _u_matrix(l3)
    Cr = np.einsum('Km,kij,Ii,Jj->KIJ', U3, Cc, U1.conj(), U2.conj(), optimize=True)
    if np.abs(Cr.imag).max() > np.abs(Cr.real).max():
        return np.ascontiguousarray(Cr.imag)
    return np.ascontiguousarray(Cr.real)


def _build_terms():
    couplings = []
    for lout in range(_LMAX_OUT + 1):
        for l1 in range(_LMAX_IN + 1):
            for l2 in range(_LMAX_IN + 1):
                if abs(l1 - l2) <= lout <= l1 + l2:
                    deg = min(_METADATA_IN[l1], _METADATA_IN[l2])
                    if deg > 0:
                        couplings.append((lout, l1, l2, deg))
    metadata_cg = [0] * (_LMAX_OUT + 1)
    metadata_out = [0] * (_LMAX_OUT + 1)
    for lo, _, _, d in couplings:
        metadata_cg[lo] += d
        metadata_out[lo] = max(metadata_out[lo], d)
    base = np.concatenate([[0], np.cumsum(metadata_cg)[:-1]]).astype(int)
    within = [0] * (_LMAX_OUT + 1)
    terms = []
    for lo, l1, l2, deg in couplings:
        C = _real_cg(l1, l2, lo)
        nz = []
        for k in range(C.shape[0]):
            for i in range(C.shape[1]):
                for j in range(C.shape[2]):
                    v = float(C[k, i, j])
                    if abs(v) > 1e-14:
                        nz.append((k, i, j, v))
        mc_off = int(base[lo]) + within[lo]
        terms.append((lo, l1, l2, deg, mc_off, nz))
        within[lo] += deg
    return terms, metadata_out


_TERMS, _META_OUT = _build_terms()
_DIM_IN = sum((2 * l + 1) * n for l, n in enumerate(_METADATA_IN))
_DIM_OUT = sum((2 * lo + 1) * _META_OUT[lo] for lo in range(_LMAX_OUT + 1))


def _body(x_ref, kc_ref, mc_ref, o_ref):
    xt = x_ref[...].T
    kc = kc_ref[...]
    mc = mc_ref[...]
    nb = xt.shape[1]

    def xseg(l, m, w):
        base = _IN_OFF[l] + m * _METADATA_IN[l]
        return xt[base:base + w, :]

    prods = {}

    def prod(l1, l2, i, j, w):
        key = (l1, l2, i, j) if (l1, l2, i, j) <= (l2, l1, j, i) else (l2, l1, j, i)
        if key not in prods:
            prods[key] = xseg(key[0], key[2], w) * xseg(key[1], key[3], w)
        return prods[key]

    acc = {}

    def add(lo, k, w, arr):
        d = acc.setdefault((lo, k), {})
        d[w] = d[w] + arr if w in d else arr

    ch = 0
    for l, nc in enumerate(_METADATA_IN):
        cp = min(nc, _META_OUT[l])
        kcv = kc[ch:ch + cp, :]
        for m in range(2 * l + 1):
            add(l, m, cp, xseg(l, m, cp) * kcv)
        ch += nc

    for (lo, l1, l2, deg, mc_off, nzs) in _TERMS:
        for (k, i, j, v) in nzs:
            tv = mc[mc_off:mc_off + deg, :] * (0.5 * v)
            add(lo, k, deg, prod(l1, l2, i, j, deg) * tv)

    blocks = []
    for lo in range(_LMAX_OUT + 1):
        W = _META_OUT[lo]
        for k in range(2 * lo + 1):
            d = acc.get((lo, k), {})
            widths = sorted(d, reverse=True)
            if widths and widths[0] == W:
                cur = d[W]
                widths = widths[1:]
            else:
                cur = jnp.zeros((W, nb), xt.dtype)
            for w in widths:
                cur = jnp.concatenate([cur[:w, :] + d[w], cur[w:, :]], axis=0)
            blocks.append(cur)
    o_ref[...] = jnp.concatenate(blocks, axis=0).T


def kernel(x, keep_coeff, mix_coeff):
    n = x.shape[0]
    kc = keep_coeff.reshape(-1, 1)
    mc = mix_coeff.reshape(-1, 1)
    NB = 1024
    grid = (n // NB,)
    out = pl.pallas_call(
        _body,
        grid=grid,
        in_specs=[
            pl.BlockSpec((NB, _DIM_IN), lambda i: (i, 0)),
            pl.BlockSpec((224, 1), lambda i: (0, 0)),
            pl.BlockSpec((864, 1), lambda i: (0, 0)),
        ],
        out_specs=pl.BlockSpec((NB, _DIM_OUT), lambda i: (i, 0)),
        out_shape=jax.ShapeDtypeStruct((n, _DIM_OUT), x.dtype),
    )(x, kc, mc)
    return out


# speedup vs baseline: 8.8051x; 1.0363x over previous
import numpy as np
import jax
import jax.numpy as jnp
from jax.experimental import pallas as pl
from jax.experimental.pallas import tpu as pltpu
from math import factorial, sqrt

_METADATA_IN = [128, 64, 32]
_LMAX_IN = 2
_LMAX_OUT = 4
_IN_OFF = [0, 128, 320]


def _cg_complex(j1, m1, j2, m2, j3, m3):
    if m1 + m2 != m3:
        return 0.0
    if not (abs(j1 - j2) <= j3 <= j1 + j2):
        return 0.0
    f = factorial
    pre = ((2 * j3 + 1) * f(j1 + j2 - j3) * f(j1 - j2 + j3) * f(-j1 + j2 + j3) / f(j1 + j2 + j3 + 1)) ** 0.5
    pre *= (f(j3 + m3) * f(j3 - m3) * f(j1 + m1) * f(j1 - m1) * f(j2 + m2) * f(j2 - m2)) ** 0.5
    kmin = max(0, j2 - j3 - m1, j1 - j3 + m2)
    kmax = min(j1 + j2 - j3, j1 - m1, j2 + m2)
    s = 0.0
    for k in range(kmin, kmax + 1):
        s += (-1) ** k / (f(k) * f(j1 + j2 - j3 - k) * f(j1 - m1 - k) * f(j2 + m2 - k) * f(j3 - j2 + m1 + k) * f(j3 - j1 - m2 + k))
    return pre * s


def _u_matrix(l):
    U = np.zeros((2 * l + 1, 2 * l + 1), dtype=np.complex128)
    U[l, l] = 1.0
    for m in range(1, l + 1):
        U[l + m, l + m] = (-1) ** m / sqrt(2.0)
        U[l + m, l - m] = 1.0 / sqrt(2.0)
        U[l - m, l - m] = 1j / sqrt(2.0)
        U[l - m, l + m] = -1j * (-1) ** m / sqrt(2.0)
    return U


def _real_cg(l1, l2, l3):
    Cc = np.zeros((2 * l3 + 1, 2 * l1 + 1, 2 * l2 + 1), dtype=np.complex128)
    for m3 in range(-l3, l3 + 1):
        for m1 in range(-l1, l1 + 1):
            m2 = m3 - m1
            if abs(m2) <= l2:
                Cc[m3 + l3, m1 + l1, m2 + l2] = _cg_complex(l1, m1, l2, m2, l3, m3)
    U1, U2, U3 = _u_matrix(l1), _u_matrix(l2), _u_matrix(l3)
    Cr = np.einsum('Km,kij,Ii,Jj->KIJ', U3, Cc, U1.conj(), U2.conj(), optimize=True)
    if np.abs(Cr.imag).max() > np.abs(Cr.real).max():
        return np.ascontiguousarray(Cr.imag)
    return np.ascontiguousarray(Cr.real)


def _build_terms():
    couplings = []
    for lout in range(_LMAX_OUT + 1):
        for l1 in range(_LMAX_IN + 1):
            for l2 in range(_LMAX_IN + 1):
                if abs(l1 - l2) <= lout <= l1 + l2:
                    deg = min(_METADATA_IN[l1], _METADATA_IN[l2])
                    if deg > 0:
                        couplings.append((lout, l1, l2, deg))
    metadata_cg = [0] * (_LMAX_OUT + 1)
    metadata_out = [0] * (_LMAX_OUT + 1)
    for lo, _, _, d in couplings:
        metadata_cg[lo] += d
        metadata_out[lo] = max(metadata_out[lo], d)
    base = np.concatenate([[0], np.cumsum(metadata_cg)[:-1]]).astype(int)
    within = [0] * (_LMAX_OUT + 1)
    terms = []
    for lo, l1, l2, deg in couplings:
        C = _real_cg(l1, l2, lo)
        nz = []
        for k in range(C.shape[0]):
            for i in range(C.shape[1]):
                for j in range(C.shape[2]):
                    v = float(C[k, i, j])
                    if abs(v) > 1e-14:
                        nz.append((k, i, j, v))
        mc_off = int(base[lo]) + within[lo]
        terms.append((lo, l1, l2, deg, mc_off, nz))
        within[lo] += deg
    return terms, metadata_out


_TERMS, _META_OUT = _build_terms()
_DIM_IN = sum((2 * l + 1) * n for l, n in enumerate(_METADATA_IN))
_DIM_OUT = sum((2 * lo + 1) * _META_OUT[lo] for lo in range(_LMAX_OUT + 1))


def _body(x_ref, kc_ref, mc_ref, o_ref):
    xt = x_ref[...].T
    kc = kc_ref[...]
    mc = mc_ref[...]
    nb = xt.shape[1]

    def xseg(l, m, w):
        base = _IN_OFF[l] + m * _METADATA_IN[l]
        return xt[base:base + w, :]

    prods = {}

    def prod(l1, l2, i, j, w):
        key = (l1, l2, i, j) if (l1, l2, i, j) <= (l2, l1, j, i) else (l2, l1, j, i)
        if key not in prods:
            prods[key] = xseg(key[0], key[2], w) * xseg(key[1], key[3], w)
        return prods[key]

    acc = {}

    def add(lo, k, w, arr):
        d = acc.setdefault((lo, k), {})
        d[w] = d[w] + arr if w in d else arr

    ch = 0
    for l, nc in enumerate(_METADATA_IN):
        cp = min(nc, _META_OUT[l])
        kcv = kc[ch:ch + cp, :]
        for m in range(2 * l + 1):
            add(l, m, cp, xseg(l, m, cp) * kcv)
        ch += nc

    for (lo, l1, l2, deg, mc_off, nzs) in _TERMS:
        for (k, i, j, v) in nzs:
            tv = mc[mc_off:mc_off + deg, :] * (0.5 * v)
            add(lo, k, deg, prod(l1, l2, i, j, deg) * tv)

    blocks = []
    for lo in range(_LMAX_OUT + 1):
        W = _META_OUT[lo]
        for k in range(2 * lo + 1):
            d = acc.get((lo, k), {})
            widths = sorted(d, reverse=True)
            if widths and widths[0] == W:
                cur = d[W]
                widths = widths[1:]
            else:
                cur = jnp.zeros((W, nb), xt.dtype)
            for w in widths:
                cur = jnp.concatenate([cur[:w, :] + d[w], cur[w:, :]], axis=0)
            blocks.append(cur)
    o_ref[...] = jnp.concatenate(blocks, axis=0).T


def kernel(x, keep_coeff, mix_coeff):
    n = x.shape[0]
    kc = keep_coeff.reshape(-1, 1)
    mc = mix_coeff.reshape(-1, 1)
    NB = 2048
    grid = (n // NB,)
    out = pl.pallas_call(
        _body,
        grid=grid,
        compiler_params=pltpu.CompilerParams(
            dimension_semantics=("arbitrary",),
        ),
        in_specs=[
            pl.BlockSpec((NB, _DIM_IN), lambda i: (i, 0)),
            pl.BlockSpec((224, 1), lambda i: (0, 0)),
            pl.BlockSpec((864, 1), lambda i: (0, 0)),
        ],
        out_specs=pl.BlockSpec((NB, _DIM_OUT), lambda i: (i, 0)),
        out_shape=jax.ShapeDtypeStruct((n, _DIM_OUT), x.dtype),
    )(x, kc, mc)
    return out
